# baseline (speedup 1.0000x reference)
import functools

import jax
import jax.numpy as jnp
from jax import lax
from jax.experimental import pallas as pl
from jax.experimental.pallas import tpu as pltpu
from jax.experimental.pallas import tpu_sc as plsc


def _sc_gather_wide(idx, table128):
    (B,) = idx.shape
    _, DW = table128.shape
    info = plsc.get_sparse_core_info()
    nc, ns = info.num_cores, info.num_subcores
    nw = nc * ns
    b_per_w = B // nw

    mesh = plsc.VectorSubcoreMesh(core_axis_name="c", subcore_axis_name="s")

    @functools.partial(
        pl.kernel,
        mesh=mesh,
        out_type=jax.ShapeDtypeStruct((B, DW), jnp.float32),
        scratch_types=[
            pltpu.VMEM((b_per_w,), jnp.int32),
            pltpu.VMEM((b_per_w,), jnp.int32),
            pltpu.VMEM((b_per_w, DW), jnp.float32),
            pltpu.SemaphoreType.DMA,
        ],
    )
    def gather_kernel(idx_hbm, table_hbm, out_hbm, idx_v, ridx_v, rows_v, sem):
        wid = lax.axis_index("s") * nc + lax.axis_index("c")
        base = wid * b_per_w
        pltpu.sync_copy(idx_hbm.at[pl.ds(base, b_per_w)], idx_v)
        for k in range(b_per_w // 16):
            sl = pl.ds(k * 16, 16)
            ridx_v[sl] = lax.shift_right_logical(idx_v[sl], 2)
        pltpu.async_copy(table_hbm.at[ridx_v], rows_v, sem).wait()
        pltpu.sync_copy(rows_v, out_hbm.at[pl.ds(base, b_per_w)])

    return gather_kernel(idx, table128)


_V_BLK = 512


def _tc_head(tok128, idxcol, pos_rep, W, b2):
    M = tok128.shape[0]
    D = pos_rep.shape[1]
    V = W.shape[0]
    nv = pl.cdiv(V, _V_BLK)

    def head_kernel(tok_ref, idx_ref, pos_ref, w_ref, b_ref, out_ref, h_ref):
        @pl.when(pl.program_id(0) == 0)
        def _():
            q = idx_ref[...] & 3
            h = pos_ref[...]
            for k in range(4):
                h = h + jnp.where(q == k, tok_ref[:, k * D:(k + 1) * D], 0.0)
            h_ref[...] = h

        acc = lax.dot_general(
            h_ref[...], w_ref[...], (((1,), (1,)), ((), ())),
            preferred_element_type=jnp.float32,
        )
        out_ref[...] = acc + b_ref[...]

    return pl.pallas_call(
        head_kernel,
        grid=(nv,),
        in_specs=[
            pl.BlockSpec((M, 4 * D), lambda j: (0, 0)),
            pl.BlockSpec((M, 1), lambda j: (0, 0)),
            pl.BlockSpec((M, D), lambda j: (0, 0)),
            pl.BlockSpec((_V_BLK, D), lambda j: (j, 0)),
            pl.BlockSpec((1, _V_BLK), lambda j: (0, j)),
        ],
        out_specs=pl.BlockSpec((M, _V_BLK), lambda j: (0, j)),
        out_shape=jax.ShapeDtypeStruct((M, V), jnp.float32),
        scratch_shapes=[pltpu.VMEM((M, D), jnp.float32)],
    )(tok128, idxcol, pos_rep, W, b2)


def kernel(x, token_table, pos_table, W, b):
    B, T = x.shape
    V, D = token_table.shape
    idx = x.reshape(-1).astype(jnp.int32)
    table128 = token_table.reshape(V * D // 128, 128)
    tok128 = _sc_gather_wide(idx, table128)
    pos_rep = jnp.tile(pos_table, (B, 1))
    out2d = _tc_head(tok128, idx.reshape(-1, 1), pos_rep, W, b.reshape(1, -1))
    return out2d.reshape(B, T, -1)


# speedup vs baseline: 2.2779x; 2.2779x over previous
import functools

import jax
import jax.numpy as jnp
from jax import lax
from jax.experimental import pallas as pl
from jax.experimental.pallas import tpu as pltpu
from jax.experimental.pallas import tpu_sc as plsc


def _sc_gather_wide(idx, table128):
    (B,) = idx.shape
    _, DW = table128.shape
    info = plsc.get_sparse_core_info()
    nc, ns = info.num_cores, info.num_subcores
    nw = nc * ns
    b_per_w = B // nw

    mesh = plsc.VectorSubcoreMesh(core_axis_name="c", subcore_axis_name="s")

    @functools.partial(
        pl.kernel,
        mesh=mesh,
        out_type=jax.ShapeDtypeStruct((B, DW), jnp.float32),
        scratch_types=[
            pltpu.VMEM((b_per_w,), jnp.int32),
            pltpu.VMEM((b_per_w,), jnp.int32),
            pltpu.VMEM((b_per_w, DW), jnp.float32),
            pltpu.SemaphoreType.DMA,
        ],
    )
    def gather_kernel(idx_hbm, table_hbm, out_hbm, idx_v, ridx_v, rows_v, sem):
        wid = lax.axis_index("s") * nc + lax.axis_index("c")
        base = wid * b_per_w
        pltpu.sync_copy(idx_hbm.at[pl.ds(base, b_per_w)], idx_v)
        for k in range(b_per_w // 16):
            sl = pl.ds(k * 16, 16)
            ridx_v[sl] = lax.shift_right_logical(idx_v[sl], 2)
        pltpu.async_copy(table_hbm.at[ridx_v], rows_v, sem).wait()
        pltpu.sync_copy(rows_v, out_hbm.at[pl.ds(base, b_per_w)])

    return gather_kernel(idx, table128)


_V_BLK = 1000


def _tc_head(tok128, idxcol, pos_rep, W, bcol, T):
    M = tok128.shape[0]
    D = pos_rep.shape[1]
    V = W.shape[0]
    NB = M // T
    nv = V // _V_BLK

    def head_kernel(tok_ref, idx_ref, pos_ref, w_ref, b_ref, out_ref, h_ref):
        @pl.when(pl.program_id(0) == 0)
        def _():
            q = idx_ref[...] & 3
            h = pos_ref[...]
            for k in range(4):
                h = h + jnp.where(q == k, tok_ref[:, k * D:(k + 1) * D], 0.0)
            h_ref[...] = h

        w = w_ref[...]
        bias = b_ref[...]
        for t in range(T):
            acc = lax.dot_general(
                w, h_ref[pl.ds(t * NB, NB), :], (((1,), (1,)), ((), ())),
                preferred_element_type=jnp.float32,
            )
            out_ref[t] = acc + bias

    return pl.pallas_call(
        head_kernel,
        grid=(nv,),
        in_specs=[
            pl.BlockSpec((M, 4 * D), lambda j: (0, 0)),
            pl.BlockSpec((M, 1), lambda j: (0, 0)),
            pl.BlockSpec((M, D), lambda j: (0, 0)),
            pl.BlockSpec((_V_BLK, D), lambda j: (j, 0)),
            pl.BlockSpec((_V_BLK, 1), lambda j: (j, 0)),
        ],
        out_specs=pl.BlockSpec((T, _V_BLK, NB), lambda j: (0, j, 0)),
        out_shape=jax.ShapeDtypeStruct((T, V, NB), jnp.float32),
        scratch_shapes=[pltpu.VMEM((M, D), jnp.float32)],
    )(tok128, idxcol, pos_rep, W, bcol)


def kernel(x, token_table, pos_table, W, b):
    B, T = x.shape
    V, D = token_table.shape
    idx = x.T.reshape(-1).astype(jnp.int32)
    table128 = token_table.reshape(V * D // 128, 128)
    tok128 = _sc_gather_wide(idx, table128)
    pos_rep = jnp.repeat(pos_table, B, axis=0)
    outT = _tc_head(tok128, idx.reshape(-1, 1), pos_rep, W, b.reshape(-1, 1), T)
    return outT.transpose(2, 0, 1)


# speedup vs baseline: 2.3112x; 1.0146x over previous
import functools

import jax
import jax.numpy as jnp
from jax import lax
from jax.experimental import pallas as pl
from jax.experimental.pallas import tpu as pltpu
from jax.experimental.pallas import tpu_sc as plsc


def _sc_gather_wide(idx, table128):
    (B,) = idx.shape
    _, DW = table128.shape
    info = plsc.get_sparse_core_info()
    nc, ns = info.num_cores, info.num_subcores
    nw = nc * ns
    b_per_w = B // nw

    mesh = plsc.VectorSubcoreMesh(core_axis_name="c", subcore_axis_name="s")

    @functools.partial(
        pl.kernel,
        mesh=mesh,
        out_type=jax.ShapeDtypeStruct((B, DW), jnp.float32),
        scratch_types=[
            pltpu.VMEM((b_per_w,), jnp.int32),
            pltpu.VMEM((b_per_w,), jnp.int32),
            pltpu.VMEM((b_per_w, DW), jnp.float32),
            pltpu.SemaphoreType.DMA,
        ],
    )
    def gather_kernel(idx_hbm, table_hbm, out_hbm, idx_v, ridx_v, rows_v, sem):
        wid = lax.axis_index("s") * nc + lax.axis_index("c")
        base = wid * b_per_w
        pltpu.sync_copy(idx_hbm.at[pl.ds(base, b_per_w)], idx_v)
        for k in range(b_per_w // 16):
            sl = pl.ds(k * 16, 16)
            ridx_v[sl] = lax.shift_right_logical(idx_v[sl], 2)
        pltpu.async_copy(table_hbm.at[ridx_v], rows_v, sem).wait()
        pltpu.sync_copy(rows_v, out_hbm.at[pl.ds(base, b_per_w)])

    return gather_kernel(idx, table128)


_V_BLK = 2000


def _tc_head(tok128, idxcol, pos_rep, W, bcol, T):
    M = tok128.shape[0]
    D = pos_rep.shape[1]
    V = W.shape[0]
    NB = M // T
    nv = V // _V_BLK

    def head_kernel(tok_ref, idx_ref, pos_ref, w_ref, b_ref, out_ref, h_ref):
        @pl.when(pl.program_id(0) == 0)
        def _():
            q = idx_ref[...] & 3
            h = pos_ref[...]
            for k in range(4):
                h = h + jnp.where(q == k, tok_ref[:, k * D:(k + 1) * D], 0.0)
            h_ref[...] = h

        w = w_ref[...]
        bias = b_ref[...]
        for t in range(T):
            acc = lax.dot_general(
                w, h_ref[pl.ds(t * NB, NB), :], (((1,), (1,)), ((), ())),
                preferred_element_type=jnp.float32,
            )
            out_ref[t] = acc + bias

    return pl.pallas_call(
        head_kernel,
        grid=(nv,),
        in_specs=[
            pl.BlockSpec((M, 4 * D), lambda j: (0, 0)),
            pl.BlockSpec((M, 1), lambda j: (0, 0)),
            pl.BlockSpec((M, D), lambda j: (0, 0)),
            pl.BlockSpec((_V_BLK, D), lambda j: (j, 0)),
            pl.BlockSpec((_V_BLK, 1), lambda j: (j, 0)),
        ],
        out_specs=pl.BlockSpec((T, _V_BLK, NB), lambda j: (0, j, 0)),
        out_shape=jax.ShapeDtypeStruct((T, V, NB), jnp.float32),
        scratch_shapes=[pltpu.VMEM((M, D), jnp.float32)],
        compiler_params=pltpu.CompilerParams(
            dimension_semantics=("arbitrary",),
        ),
    )(tok128, idxcol, pos_rep, W, bcol)


def kernel(x, token_table, pos_table, W, b):
    B, T = x.shape
    V, D = token_table.shape
    idx = x.T.reshape(-1).astype(jnp.int32)
    table128 = token_table.reshape(V * D // 128, 128)
    tok128 = _sc_gather_wide(idx, table128)
    pos_rep = jnp.repeat(pos_table, B, axis=0)
    outT = _tc_head(tok128, idx.reshape(-1, 1), pos_rep, W, b.reshape(-1, 1), T)
    return outT.transpose(2, 0, 1)


# speedup vs baseline: 2.3141x; 1.0013x over previous
import functools

import jax
import jax.numpy as jnp
from jax import lax
from jax.experimental import pallas as pl
from jax.experimental.pallas import tpu as pltpu
from jax.experimental.pallas import tpu_sc as plsc


def _sc_gather_wide(idx, table128):
    (B,) = idx.shape
    _, DW = table128.shape
    info = plsc.get_sparse_core_info()
    nc, ns = info.num_cores, info.num_subcores
    nw = nc * ns
    b_per_w = B // nw

    mesh = plsc.VectorSubcoreMesh(core_axis_name="c", subcore_axis_name="s")

    @functools.partial(
        pl.kernel,
        mesh=mesh,
        out_type=jax.ShapeDtypeStruct((B, DW), jnp.float32),
        scratch_types=[
            pltpu.VMEM((b_per_w,), jnp.int32),
            pltpu.VMEM((b_per_w,), jnp.int32),
            pltpu.VMEM((b_per_w, DW), jnp.float32),
            pltpu.SemaphoreType.DMA,
        ],
    )
    def gather_kernel(idx_hbm, table_hbm, out_hbm, idx_v, ridx_v, rows_v, sem):
        wid = lax.axis_index("s") * nc + lax.axis_index("c")
        base = wid * b_per_w
        pltpu.sync_copy(idx_hbm.at[pl.ds(base, b_per_w)], idx_v)
        for k in range(b_per_w // 16):
            sl = pl.ds(k * 16, 16)
            ridx_v[sl] = lax.shift_right_logical(idx_v[sl], 2)
        pltpu.async_copy(table_hbm.at[ridx_v], rows_v, sem).wait()
        pltpu.sync_copy(rows_v, out_hbm.at[pl.ds(base, b_per_w)])

    return gather_kernel(idx, table128)


_V_BLK = 2000


def _tc_head(tok128, idxcol, pos_rep, W, bcol, T):
    M = tok128.shape[0]
    D = pos_rep.shape[1]
    V = W.shape[0]
    NB = M // T
    nv = V // _V_BLK

    def head_kernel(tok_ref, idx_ref, pos_ref, w_ref, b_ref, out_ref, h_ref):
        @pl.when(pl.program_id(0) == 0)
        def _():
            q = idx_ref[...] & 3
            h = pos_ref[...]
            for k in range(4):
                h = h + jnp.where(q == k, tok_ref[:, k * D:(k + 1) * D], 0.0)
            h_ref[...] = h

        w = w_ref[...]
        bias = b_ref[...]
        for t in range(T):
            acc = lax.dot_general(
                w, h_ref[pl.ds(t * NB, NB), :], (((1,), (1,)), ((), ())),
                preferred_element_type=jnp.float32,
            )
            out_ref[t] = acc + bias

    return pl.pallas_call(
        head_kernel,
        grid=(nv,),
        in_specs=[
            pl.BlockSpec((M, 4 * D), lambda j: (0, 0)),
            pl.BlockSpec((M, 1), lambda j: (0, 0)),
            pl.BlockSpec((M, D), lambda j: (0, 0)),
            pl.BlockSpec((_V_BLK, D), lambda j: (j, 0)),
            pl.BlockSpec((_V_BLK, 1), lambda j: (j, 0)),
        ],
        out_specs=pl.BlockSpec((T, _V_BLK, NB), lambda j: (0, j, 0)),
        out_shape=jax.ShapeDtypeStruct((T, V, NB), jnp.float32),
        scratch_shapes=[pltpu.VMEM((M, D), jnp.float32)],
        compiler_params=pltpu.CompilerParams(
            dimension_semantics=("parallel",),
        ),
    )(tok128, idxcol, pos_rep, W, bcol)


def kernel(x, token_table, pos_table, W, b):
    B, T = x.shape
    V, D = token_table.shape
    idx = x.T.reshape(-1).astype(jnp.int32)
    table128 = token_table.reshape(V * D // 128, 128)
    tok128 = _sc_gather_wide(idx, table128)
    pos_rep = jnp.repeat(pos_table, B, axis=0)
    outT = _tc_head(tok128, idx.reshape(-1, 1), pos_rep, W, b.reshape(-1, 1), T)
    return outT.transpose(2, 0, 1)
